# Initial kernel scaffold; baseline (speedup 1.0000x reference)
import jax
import jax.numpy as jnp
from jax.experimental import pallas as pl


def kernel(feat_g):
    raise NotImplementedError("write your pallas kernel here")


# speedup vs baseline: 1.4044x; 1.4044x over previous
import jax
import jax.numpy as jnp
from jax import lax
from jax.experimental import pallas as pl
from jax.experimental.pallas import tpu as pltpu

_N_CLUSTERS = 1024
_ITERS = 10
_DIM = 256
_ROW_BLOCK = 2048


def _c2_row(c):
    ones_dim = jnp.ones((1, _DIM), jnp.float32)
    return lax.dot_general(ones_dim, c * c, (((1,), (1,)), ((), ())),
                           preferred_element_type=jnp.float32,
                           precision=lax.Precision.HIGHEST)


def _assign_block(xb, c, c2r):
    x2 = jnp.sum(xb * xb, axis=1, keepdims=True)
    xc = lax.dot_general(xb, c, (((1,), (1,)), ((), ())),
                         preferred_element_type=jnp.float32,
                         precision=lax.Precision.DEFAULT)
    d = x2 + c2r - 2.0 * xc
    dmin = jnp.min(d, axis=1, keepdims=True)
    idx = lax.broadcasted_iota(jnp.int32, d.shape, 1)
    return jnp.min(jnp.where(d == dmin, idx, _N_CLUSTERS), axis=1, keepdims=True)


def _km_kernel(x_ref, out_ref, c_ref, sums_ref, counts_ref):
    n = x_ref.shape[0]
    nblk = n // _ROW_BLOCK
    c_ref[...] = x_ref[0:_N_CLUSTERS, :]
    ones_col = jnp.ones((_ROW_BLOCK, 1), jnp.float32)

    def iter_body(it, carry):
        c = c_ref[...]
        c2r = _c2_row(c)
        sums_ref[...] = jnp.zeros_like(sums_ref)
        counts_ref[...] = jnp.zeros_like(counts_ref)

        def blk(b, carry2):
            xb = x_ref[pl.ds(b * _ROW_BLOCK, _ROW_BLOCK), :]
            assign = _assign_block(xb, c, c2r)
            idx = lax.broadcasted_iota(jnp.int32, (_ROW_BLOCK, _N_CLUSTERS), 1)
            onehot = (assign == idx).astype(jnp.float32)
            sums_ref[...] += lax.dot_general(
                onehot, xb, (((0,), (0,)), ((), ())),
                preferred_element_type=jnp.float32,
                precision=lax.Precision.HIGHEST)
            counts_ref[...] += lax.dot_general(
                onehot, ones_col, (((0,), (0,)), ((), ())),
                preferred_element_type=jnp.float32,
                precision=lax.Precision.HIGHEST)
            return carry2

        lax.fori_loop(0, nblk, blk, 0)
        counts = counts_ref[...]
        newc = sums_ref[...] / jnp.maximum(counts, 1.0)
        c_ref[...] = jnp.where(counts > 0, newc, c)
        return carry

    lax.fori_loop(0, _ITERS, iter_body, 0)

    c = c_ref[...]
    c2r = _c2_row(c)

    def final_blk(b, carry):
        xb = x_ref[pl.ds(b * _ROW_BLOCK, _ROW_BLOCK), :]
        out_ref[pl.ds(b * _ROW_BLOCK, _ROW_BLOCK), :] = _assign_block(xb, c, c2r)
        return carry

    lax.fori_loop(0, nblk, final_blk, 0)


def kernel(feat_g):
    n, dim = feat_g.shape
    preds = pl.pallas_call(
        _km_kernel,
        out_shape=jax.ShapeDtypeStruct((n, 1), jnp.int32),
        scratch_shapes=[
            pltpu.VMEM((_N_CLUSTERS, _DIM), jnp.float32),
            pltpu.VMEM((_N_CLUSTERS, _DIM), jnp.float32),
            pltpu.VMEM((_N_CLUSTERS, 1), jnp.float32),
        ],
    )(feat_g)
    return preds.reshape(n)


# speedup vs baseline: 2.2469x; 1.5998x over previous
import jax
import jax.numpy as jnp
from jax import lax
from jax.experimental import pallas as pl
from jax.experimental.pallas import tpu as pltpu

_N_CLUSTERS = 1024
_ITERS = 10
_DIM = 256
_ROW_BLOCK = 2048


def _c2_row(c):
    ones_dim = jnp.ones((1, _DIM), jnp.float32)
    return lax.dot_general(ones_dim, c * c, (((1,), (1,)), ((), ())),
                           preferred_element_type=jnp.float32,
                           precision=lax.Precision.HIGHEST)


def _assign_block(xb, c, c2r):
    x2 = jnp.sum(xb * xb, axis=1, keepdims=True)
    xc = lax.dot_general(xb, c, (((1,), (1,)), ((), ())),
                         preferred_element_type=jnp.float32,
                         precision=lax.Precision.DEFAULT)
    d = x2 + c2r - 2.0 * xc
    return jnp.argmin(d, axis=1, keepdims=True).astype(jnp.int32)


def _split3_bf16(x):
    hi = x.astype(jnp.bfloat16)
    r1 = x - hi.astype(jnp.float32)
    mid = r1.astype(jnp.bfloat16)
    lo = (r1 - mid.astype(jnp.float32)).astype(jnp.bfloat16)
    return hi, mid, lo


def _km_kernel(x_ref, out_ref, c_ref, sums_ref, counts_ref):
    n = x_ref.shape[0]
    nblk = n // _ROW_BLOCK
    c_ref[...] = x_ref[0:_N_CLUSTERS, :]
    ones_col = jnp.ones((_ROW_BLOCK, 1), jnp.bfloat16)

    def iter_body(it, carry):
        c = c_ref[...]
        c2r = _c2_row(c)
        sums_ref[...] = jnp.zeros_like(sums_ref)
        counts_ref[...] = jnp.zeros_like(counts_ref)

        def blk(b, carry2):
            xb = x_ref[pl.ds(b * _ROW_BLOCK, _ROW_BLOCK), :]
            assign = _assign_block(xb, c, c2r)
            idx = lax.broadcasted_iota(jnp.int32, (_ROW_BLOCK, _N_CLUSTERS), 1)
            onehot = (assign == idx).astype(jnp.bfloat16)
            xhi, xmid, xlo = _split3_bf16(xb)
            cdims = (((0,), (0,)), ((), ()))
            acc = lax.dot_general(onehot, xhi, cdims,
                                  preferred_element_type=jnp.float32)
            acc += lax.dot_general(onehot, xmid, cdims,
                                   preferred_element_type=jnp.float32)
            acc += lax.dot_general(onehot, xlo, cdims,
                                   preferred_element_type=jnp.float32)
            sums_ref[...] += acc
            counts_ref[...] += lax.dot_general(
                onehot, ones_col, cdims,
                preferred_element_type=jnp.float32)
            return carry2

        lax.fori_loop(0, nblk, blk, 0)
        counts = counts_ref[...]
        newc = sums_ref[...] / jnp.maximum(counts, 1.0)
        c_ref[...] = jnp.where(counts > 0, newc, c)
        return carry

    lax.fori_loop(0, _ITERS, iter_body, 0)

    c = c_ref[...]
    c2r = _c2_row(c)

    def final_blk(b, carry):
        xb = x_ref[pl.ds(b * _ROW_BLOCK, _ROW_BLOCK), :]
        out_ref[pl.ds(b * _ROW_BLOCK, _ROW_BLOCK), :] = _assign_block(xb, c, c2r)
        return carry

    lax.fori_loop(0, nblk, final_blk, 0)


def kernel(feat_g):
    n, dim = feat_g.shape
    preds = pl.pallas_call(
        _km_kernel,
        out_shape=jax.ShapeDtypeStruct((n, 1), jnp.int32),
        scratch_shapes=[
            pltpu.VMEM((_N_CLUSTERS, _DIM), jnp.float32),
            pltpu.VMEM((_N_CLUSTERS, _DIM), jnp.float32),
            pltpu.VMEM((_N_CLUSTERS, 1), jnp.float32),
        ],
    )(feat_g)
    return preds.reshape(n)


# speedup vs baseline: 2.2569x; 1.0045x over previous
import jax
import jax.numpy as jnp
from jax import lax
from jax.experimental import pallas as pl
from jax.experimental.pallas import tpu as pltpu

_N_CLUSTERS = 1024
_ITERS = 10
_DIM = 256
_ROW_BLOCK = 2048


def _c2_row(c):
    ones_dim = jnp.ones((1, _DIM), jnp.float32)
    return lax.dot_general(ones_dim, c * c, (((1,), (1,)), ((), ())),
                           preferred_element_type=jnp.float32,
                           precision=lax.Precision.HIGHEST)


def _assign_block(xb, c, c2r):
    x2 = jnp.sum(xb * xb, axis=1, keepdims=True)
    xc = lax.dot_general(xb, c, (((1,), (1,)), ((), ())),
                         preferred_element_type=jnp.float32,
                         precision=lax.Precision.DEFAULT)
    d = x2 + c2r - 2.0 * xc
    return jnp.argmin(d, axis=1, keepdims=True).astype(jnp.int32)


def _split3_bf16(x):
    hi = x.astype(jnp.bfloat16)
    r1 = x - hi.astype(jnp.float32)
    mid = r1.astype(jnp.bfloat16)
    lo = (r1 - mid.astype(jnp.float32)).astype(jnp.bfloat16)
    return hi, mid, lo


def _km_kernel(x_ref, out_ref, c_ref, sums_ref, counts_ref,
               xhi_ref, xmid_ref, xlo_ref):
    n = x_ref.shape[0]
    nblk = n // _ROW_BLOCK
    c_ref[...] = x_ref[0:_N_CLUSTERS, :]
    ones_col = jnp.ones((_ROW_BLOCK, 1), jnp.bfloat16)

    def pre_blk(b, carry):
        sl = pl.ds(b * _ROW_BLOCK, _ROW_BLOCK)
        xb = x_ref[sl, :]
        hi, mid, lo = _split3_bf16(xb)
        xhi_ref[sl, :] = hi
        xmid_ref[sl, :] = mid
        xlo_ref[sl, :] = lo
        return carry

    lax.fori_loop(0, nblk, pre_blk, 0)

    def iter_body(it, carry):
        c = c_ref[...]
        c2r = _c2_row(c)
        sums_ref[...] = jnp.zeros_like(sums_ref)
        counts_ref[...] = jnp.zeros_like(counts_ref)

        def blk(b, carry2):
            sl = pl.ds(b * _ROW_BLOCK, _ROW_BLOCK)
            assign = _assign_block(x_ref[sl, :], c, c2r)
            idx = lax.broadcasted_iota(jnp.int32, (_ROW_BLOCK, _N_CLUSTERS), 1)
            onehot = (assign == idx).astype(jnp.bfloat16)
            cdims = (((0,), (0,)), ((), ()))
            acc = lax.dot_general(onehot, xhi_ref[sl, :], cdims,
                                  preferred_element_type=jnp.float32)
            acc += lax.dot_general(onehot, xmid_ref[sl, :], cdims,
                                   preferred_element_type=jnp.float32)
            acc += lax.dot_general(onehot, xlo_ref[sl, :], cdims,
                                   preferred_element_type=jnp.float32)
            sums_ref[...] += acc
            counts_ref[...] += lax.dot_general(
                onehot, ones_col, cdims,
                preferred_element_type=jnp.float32)
            return carry2

        lax.fori_loop(0, nblk, blk, 0)
        counts = counts_ref[...]
        newc = sums_ref[...] / jnp.maximum(counts, 1.0)
        c_ref[...] = jnp.where(counts > 0, newc, c)
        return carry

    lax.fori_loop(0, _ITERS, iter_body, 0)

    c = c_ref[...]
    c2r = _c2_row(c)

    def final_blk(b, carry):
        sl = pl.ds(b * _ROW_BLOCK, _ROW_BLOCK)
        out_ref[sl, :] = _assign_block(x_ref[sl, :], c, c2r)
        return carry

    lax.fori_loop(0, nblk, final_blk, 0)


def kernel(feat_g):
    n, dim = feat_g.shape
    preds = pl.pallas_call(
        _km_kernel,
        out_shape=jax.ShapeDtypeStruct((n, 1), jnp.int32),
        scratch_shapes=[
            pltpu.VMEM((_N_CLUSTERS, _DIM), jnp.float32),
            pltpu.VMEM((_N_CLUSTERS, _DIM), jnp.float32),
            pltpu.VMEM((_N_CLUSTERS, 1), jnp.float32),
            pltpu.VMEM((n, _DIM), jnp.bfloat16),
            pltpu.VMEM((n, _DIM), jnp.bfloat16),
            pltpu.VMEM((n, _DIM), jnp.bfloat16),
        ],
        compiler_params=pltpu.CompilerParams(
            vmem_limit_bytes=100 * 1024 * 1024),
    )(feat_g)
    return preds.reshape(n)


# speedup vs baseline: 3.1282x; 1.3861x over previous
import jax
import jax.numpy as jnp
from jax import lax
from jax.experimental import pallas as pl
from jax.experimental.pallas import tpu as pltpu

_N_CLUSTERS = 1024
_ITERS = 10
_DIM = 256
_ROW_BLOCK = 1024


def _c2_row(c):
    ones_dim = jnp.ones((1, _DIM), jnp.float32)
    return lax.dot_general(ones_dim, c * c, (((1,), (1,)), ((), ())),
                           preferred_element_type=jnp.float32,
                           precision=lax.Precision.HIGHEST)


def _assign_block(xb, c, c2r):
    x2 = jnp.sum(xb * xb, axis=1, keepdims=True)
    xc = lax.dot_general(xb, c, (((1,), (1,)), ((), ())),
                         preferred_element_type=jnp.float32,
                         precision=lax.Precision.DEFAULT)
    d = x2 + c2r - 2.0 * xc
    return jnp.argmin(d, axis=1, keepdims=True).astype(jnp.int32)


def _split3_bf16(x):
    hi = x.astype(jnp.bfloat16)
    r1 = x - hi.astype(jnp.float32)
    mid = r1.astype(jnp.bfloat16)
    lo = (r1 - mid.astype(jnp.float32)).astype(jnp.bfloat16)
    return hi, mid, lo


def _km_kernel(x_ref, out_ref, c_ref, sums_ref, counts_ref,
               xhi_ref, xmid_ref, xlo_ref):
    n = x_ref.shape[0]
    nblk = n // _ROW_BLOCK
    c_ref[...] = x_ref[0:_N_CLUSTERS, :]
    ones_col = jnp.ones((_ROW_BLOCK, 1), jnp.bfloat16)

    def pre_blk(b, carry):
        sl = pl.ds(b * _ROW_BLOCK, _ROW_BLOCK)
        xb = x_ref[sl, :]
        hi, mid, lo = _split3_bf16(xb)
        xhi_ref[sl, :] = hi
        xmid_ref[sl, :] = mid
        xlo_ref[sl, :] = lo
        return carry

    lax.fori_loop(0, nblk, pre_blk, 0)

    def iter_body(it, carry):
        c = c_ref[...]
        c2r = _c2_row(c)
        sums_ref[...] = jnp.zeros_like(sums_ref)
        counts_ref[...] = jnp.zeros_like(counts_ref)

        xc0 = lax.dot_general(x_ref[pl.ds(0, _ROW_BLOCK), :], c,
                              (((1,), (1,)), ((), ())),
                              preferred_element_type=jnp.float32,
                              precision=lax.Precision.DEFAULT)
        oh0 = jnp.zeros((_ROW_BLOCK, _N_CLUSTERS), jnp.bfloat16)

        def blk(b, carry2):
            xc_b, oh_prev = carry2
            bprev = jnp.maximum(b - 1, 0)
            bcur = jnp.minimum(b, nblk - 1)
            bnext = jnp.minimum(b + 1, nblk - 1)
            slp = pl.ds(bprev * _ROW_BLOCK, _ROW_BLOCK)
            slc = pl.ds(bcur * _ROW_BLOCK, _ROW_BLOCK)
            sln = pl.ds(bnext * _ROW_BLOCK, _ROW_BLOCK)
            cdims = (((0,), (0,)), ((), ()))
            acc = lax.dot_general(oh_prev, xhi_ref[slp, :], cdims,
                                  preferred_element_type=jnp.float32)
            acc += lax.dot_general(oh_prev, xmid_ref[slp, :], cdims,
                                   preferred_element_type=jnp.float32)
            acc += lax.dot_general(oh_prev, xlo_ref[slp, :], cdims,
                                   preferred_element_type=jnp.float32)
            sums_ref[...] += acc
            counts_ref[...] += lax.dot_general(
                oh_prev, ones_col, cdims,
                preferred_element_type=jnp.float32)
            xc_next = lax.dot_general(x_ref[sln, :], c,
                                      (((1,), (1,)), ((), ())),
                                      preferred_element_type=jnp.float32,
                                      precision=lax.Precision.DEFAULT)
            xb = x_ref[slc, :]
            x2 = jnp.sum(xb * xb, axis=1, keepdims=True)
            d = x2 + c2r - 2.0 * xc_b
            assign = jnp.argmin(d, axis=1, keepdims=True).astype(jnp.int32)
            idx = lax.broadcasted_iota(jnp.int32, (_ROW_BLOCK, _N_CLUSTERS), 1)
            onehot = (assign == idx).astype(jnp.bfloat16)
            return (xc_next, onehot)

        lax.fori_loop(0, nblk + 1, blk, (xc0, oh0))
        counts = counts_ref[...]
        newc = sums_ref[...] / jnp.maximum(counts, 1.0)
        c_ref[...] = jnp.where(counts > 0, newc, c)
        return carry

    lax.fori_loop(0, _ITERS, iter_body, 0)

    c = c_ref[...]
    c2r = _c2_row(c)

    def final_blk(b, carry):
        sl = pl.ds(b * _ROW_BLOCK, _ROW_BLOCK)
        assign = _assign_block(x_ref[sl, :], c, c2r)
        out_ref[sl] = jnp.squeeze(assign, -1)
        return carry

    lax.fori_loop(0, nblk, final_blk, 0)


def kernel(feat_g):
    n, dim = feat_g.shape
    out = pl.pallas_call(
        _km_kernel,
        out_shape=jax.ShapeDtypeStruct((n,), jnp.int32),
        scratch_shapes=[
            pltpu.VMEM((_N_CLUSTERS, _DIM), jnp.float32),
            pltpu.VMEM((_N_CLUSTERS, _DIM), jnp.float32),
            pltpu.VMEM((_N_CLUSTERS, 1), jnp.float32),
            pltpu.VMEM((n, _DIM), jnp.bfloat16),
            pltpu.VMEM((n, _DIM), jnp.bfloat16),
            pltpu.VMEM((n, _DIM), jnp.bfloat16),
        ],
        compiler_params=pltpu.CompilerParams(
            vmem_limit_bytes=63 * 1024 * 1024),
    )(feat_g)
    return out
